# Initial kernel scaffold; baseline (speedup 1.0000x reference)
import jax
import jax.numpy as jnp
from jax.experimental import pallas as pl


def kernel(node_type, edge_type, edge_index, edge_label_index, node_emb, edge_emb, Ws1, bs1, Wn1, bn1, Ws2, bs2, Wn2, bn2, Ws3, bs3, Wn3, bn3, hW1, hb1, hW2, hb2):
    raise NotImplementedError("write your pallas kernel here")


# speedup vs baseline: 7.4116x; 7.4116x over previous
import jax
import jax.numpy as jnp
from jax import lax
from jax.experimental import pallas as pl
from jax.experimental.pallas import tpu as pltpu
from jax.experimental.pallas import tpu_sc as plsc

_N = 10000
_E = 320000
_D = 128
_B = 1024

_NC = 2
_NS = 16
_NW = _NC * _NS
_EPW = _E // _NW
_NP = 10240
_RPT = _NP // _NS

_SS_CH = 100
_SS_K = _EPW // _SS_CH

_HG_PW = 2 * _B // _NW

_sc_mesh = plsc.VectorSubcoreMesh(core_axis_name="c", subcore_axis_name="s")


def _segsum_body(x_hbm, src_hbm, dst_hbm, z_hbm, out_hbm,
                 src_v, dst_v, rows_v, acc_sh, sem):
    c = lax.axis_index("c")
    s = lax.axis_index("s")
    wid = c * _NS + s
    pltpu.sync_copy(src_hbm.at[wid], src_v)
    pltpu.sync_copy(dst_hbm.at[wid], dst_v)
    pltpu.sync_copy(z_hbm, acc_sh.at[pl.ds(s * _RPT, _RPT)])
    plsc.subcore_barrier()

    def chunk(j, carry):
        pltpu.async_copy(x_hbm.at[src_v.at[j]], rows_v, sem).wait()
        pltpu.sync_copy(rows_v, acc_sh.at[dst_v.at[j]], add=True)
        return carry

    lax.fori_loop(0, _SS_K, chunk, 0)
    plsc.subcore_barrier()
    pltpu.sync_copy(acc_sh.at[pl.ds(s * _RPT, _RPT)],
                    out_hbm.at[c, pl.ds(s * _RPT, _RPT)])


_segsum_call = pl.kernel(
    _segsum_body,
    out_type=jax.ShapeDtypeStruct((_NC, _NP, _D), jnp.float32),
    mesh=_sc_mesh,
    scratch_types=[
        pltpu.VMEM((_SS_K, _SS_CH), jnp.int32),
        pltpu.VMEM((_SS_K, _SS_CH), jnp.int32),
        pltpu.VMEM((_SS_CH, _D), jnp.float32),
        pltpu.VMEM_SHARED((_NP, _D), jnp.float32),
        pltpu.SemaphoreType.DMA,
    ],
)


def _hgather_body(x_hbm, idx_hbm, out_hbm, idx_v, rows_v, sem):
    c = lax.axis_index("c")
    s = lax.axis_index("s")
    wid = c * _NS + s
    pltpu.sync_copy(idx_hbm.at[wid], idx_v)
    pltpu.async_copy(x_hbm.at[idx_v], rows_v, sem).wait()
    pltpu.sync_copy(rows_v, out_hbm.at[pl.ds(wid * _HG_PW, _HG_PW)])


_hgather_call = pl.kernel(
    _hgather_body,
    out_type=jax.ShapeDtypeStruct((2 * _B, _D), jnp.float32),
    mesh=_sc_mesh,
    scratch_types=[
        pltpu.VMEM((_HG_PW,), jnp.int32),
        pltpu.VMEM((_HG_PW, _D), jnp.float32),
        pltpu.SemaphoreType.DMA,
    ],
)


_BLK = 1000


def _layer1_body(nt_ref, cp_ref, emb_ref, ws_ref, wn_ref, b_ref,
                 x_out, inv_out):
    cb = cp_ref[0] + cp_ref[1]
    deg = cb[:, 16:17]
    inv = 1.0 / jnp.maximum(deg, 1.0)
    emb = emb_ref[...]
    onehot = (nt_ref[...] == lax.broadcasted_iota(jnp.int32, (_BLK, _D), 1)
              ).astype(jnp.float32)
    x0 = jnp.dot(onehot, emb, preferred_element_type=jnp.float32)
    mean = jnp.dot(cb, emb, preferred_element_type=jnp.float32) * inv
    out = (jnp.dot(x0, ws_ref[...], preferred_element_type=jnp.float32)
           + jnp.dot(mean, wn_ref[...], preferred_element_type=jnp.float32)
           + b_ref[...])
    x_out[...] = jnp.maximum(out, 0.0)
    inv_out[...] = inv


def _layer1_call(nt2d, cp, emb16, ws, wn, b):
    return pl.pallas_call(
        _layer1_body,
        grid=(_N // _BLK,),
        in_specs=[
            pl.BlockSpec((_BLK, 1), lambda i: (i, 0)),
            pl.BlockSpec((_NC, _BLK, _D), lambda i: (0, i, 0)),
            pl.BlockSpec((_D, _D), lambda i: (0, 0)),
            pl.BlockSpec((_D, _D), lambda i: (0, 0)),
            pl.BlockSpec((_D, _D), lambda i: (0, 0)),
            pl.BlockSpec((1, _D), lambda i: (0, 0)),
        ],
        out_specs=[
            pl.BlockSpec((_BLK, _D), lambda i: (i, 0)),
            pl.BlockSpec((_BLK, 1), lambda i: (i, 0)),
        ],
        out_shape=[
            jax.ShapeDtypeStruct((_N, _D), jnp.float32),
            jax.ShapeDtypeStruct((_N, 1), jnp.float32),
        ],
    )(nt2d, cp, emb16, ws, wn, b)


def _dense_body(x_ref, pp_ref, inv_ref, ws_ref, wn_ref, b_ref, x_out):
    mean = (pp_ref[0] + pp_ref[1]) * inv_ref[...]
    out = (jnp.dot(x_ref[...], ws_ref[...], preferred_element_type=jnp.float32)
           + jnp.dot(mean, wn_ref[...], preferred_element_type=jnp.float32)
           + b_ref[...])
    x_out[...] = jnp.maximum(out, 0.0)


def _dense_call(x, pp, inv, ws, wn, b):
    return pl.pallas_call(
        _dense_body,
        grid=(_N // _BLK,),
        in_specs=[
            pl.BlockSpec((_BLK, _D), lambda i: (i, 0)),
            pl.BlockSpec((_NC, _BLK, _D), lambda i: (0, i, 0)),
            pl.BlockSpec((_BLK, 1), lambda i: (i, 0)),
            pl.BlockSpec((_D, _D), lambda i: (0, 0)),
            pl.BlockSpec((_D, _D), lambda i: (0, 0)),
            pl.BlockSpec((1, _D), lambda i: (0, 0)),
        ],
        out_specs=pl.BlockSpec((_BLK, _D), lambda i: (i, 0)),
        out_shape=jax.ShapeDtypeStruct((_N, _D), jnp.float32),
    )(x, pp, inv, ws, wn, b)


def _head_body(h_ref, w1a_ref, w1b_ref, b1_ref, w2_ref, b2_ref, out_ref):
    h = h_ref[...]
    h0 = h[:_B]
    h1 = h[_B:]
    z = (jnp.dot(h0, w1a_ref[...], preferred_element_type=jnp.float32)
         + jnp.dot(h1, w1b_ref[...], preferred_element_type=jnp.float32)
         + b1_ref[...])
    z = jnp.maximum(z, 0.0)
    out_ref[...] = jnp.dot(z, w2_ref[...],
                           preferred_element_type=jnp.float32) + b2_ref[...]


def _head_call(h, w1a, w1b, b1, w2, b2):
    return pl.pallas_call(
        _head_body,
        out_shape=jax.ShapeDtypeStruct((_B, 1), jnp.float32),
    )(h, w1a, w1b, b1, w2, b2)


def kernel(node_type, edge_type, edge_index, edge_label_index,
           node_emb, edge_emb,
           Ws1, bs1, Wn1, bn1,
           Ws2, bs2, Wn2, bn2,
           Ws3, bs3, Wn3, bn3,
           hW1, hb1, hW2, hb2):
    del edge_type, edge_emb

    src = edge_index[0]
    dst = edge_index[1]
    src_ss = src.reshape(_NW, _SS_K, _SS_CH)
    dst_ss = dst.reshape(_NW, _SS_K, _SS_CH)

    zD = jnp.zeros((_RPT, _D), jnp.float32)
    emb128 = jnp.zeros((_D, _D), jnp.float32).at[:4].set(node_emb)
    nt2d = node_type.reshape(_N, 1)
    col = jnp.arange(_D, dtype=jnp.int32)[None, :]
    t1 = jnp.where(col == nt2d, 1.0, 0.0) + jnp.where(col == 16, 1.0, 0.0)
    t1 = t1.astype(jnp.float32)

    cp = _segsum_call(t1, src_ss, dst_ss, zD)
    x1, inv = _layer1_call(nt2d, cp, emb128,
                           Ws1, Wn1, (bs1 + bn1).reshape(1, _D))

    p2 = _segsum_call(x1, src_ss, dst_ss, zD)
    x2 = _dense_call(x1, p2, inv, Ws2, Wn2, (bs2 + bn2).reshape(1, _D))
    p3 = _segsum_call(x2, src_ss, dst_ss, zD)
    x3 = _dense_call(x2, p3, inv, Ws3, Wn3, (bs3 + bn3).reshape(1, _D))

    h = _hgather_call(x3, edge_label_index.reshape(_NW, _HG_PW))
    out = _head_call(h, hW1[:_D], hW1[_D:], hb1.reshape(1, _D),
                     hW2, hb2.reshape(1, 1))
    return out
